# baseline (speedup 1.0000x reference)
import functools

import jax
import jax.numpy as jnp
from jax import lax
from jax.experimental import pallas as pl
from jax.experimental.pallas import tpu as pltpu
from jax.experimental.pallas import tpu_sc as plsc

N_NODE = 10000
E = 160000
D = 128
F = 128
DO = 64

NC, NS = 2, 16
NW = NC * NS
RNG = 320
NP = NW * RNG
C = 800
NV = C // 16
NCHUNK = E // C
G = 32
LCAP = C + G + 16

_NEG_INF = float('-inf')
_POS_INF = float('inf')


def _pre_body(x_ref, w_ref, b_ref, ws_ref, bs_ref, xp_ref, p_ref):
    xp = jnp.dot(x_ref[...], w_ref[...], preferred_element_type=jnp.float32)
    xp = xp + b_ref[...]
    xp_ref[...] = xp
    p_ref[...] = jnp.dot(xp, ws_ref[...],
                         preferred_element_type=jnp.float32) + bs_ref[...]


def _pre(x, w, b, ws, bs):
    return pl.pallas_call(
        _pre_body,
        out_shape=(
            jax.ShapeDtypeStruct((N_NODE, F), jnp.float32),
            jax.ShapeDtypeStruct((N_NODE, 1), jnp.float32),
        ),
    )(x, w, b, ws, bs)


def _post_body(x_ref, xp_ref, csum_ref, cmax_ref, satt_ref, cnt_ref,
               amax_ref, amin_ref, w_ref, b_ref, h_ref, *, self_first):
    x = x_ref[...]
    xp = xp_ref[...]
    satt_ref_v = satt_ref[...]
    cnt = cnt_ref[...]
    amax_v = amax_ref[...]
    amin_v = amin_ref[...]
    has = cnt > 0.0
    inv = 1.0 / jnp.maximum(cnt, 1.0)
    mean_self = xp * (satt_ref_v * inv)
    mean_cross = csum_ref[...] * inv
    ms = jnp.where(xp >= 0.0, xp * amax_v, xp * amin_v)
    max_self = jnp.maximum(jnp.where(has, ms, 0.0), 0.0)
    max_cross = jnp.maximum(jnp.where(has, cmax_ref[...], 0.0), 0.0)
    if self_first:
        h = jnp.concatenate(
            [x, xp, mean_self, mean_cross, max_self, max_cross], axis=-1)
    else:
        h = jnp.concatenate(
            [x, xp, mean_cross, mean_self, max_cross, max_self], axis=-1)
    out = jnp.dot(h, w_ref[...], preferred_element_type=jnp.float32)
    h_ref[...] = jnp.maximum(out + b_ref[...], 0.0)


def _post(x, xp, csum, cmax, satt, cnt, amax, amin, w, b, self_first):
    blk = 2000
    grid = N_NODE // blk
    rs = lambda i: (i, 0)
    full = lambda i: (0, 0)
    return pl.pallas_call(
        functools.partial(_post_body, self_first=self_first),
        grid=(grid,),
        in_specs=[
            pl.BlockSpec((blk, D), rs),
            pl.BlockSpec((blk, F), rs),
            pl.BlockSpec((blk, F), rs),
            pl.BlockSpec((blk, F), rs),
            pl.BlockSpec((blk, 1), rs),
            pl.BlockSpec((blk, 1), rs),
            pl.BlockSpec((blk, 1), rs),
            pl.BlockSpec((blk, 1), rs),
            pl.BlockSpec((D + F + 4 * F, DO), full),
            pl.BlockSpec((1, DO), full),
        ],
        out_specs=pl.BlockSpec((blk, DO), rs),
        out_shape=jax.ShapeDtypeStruct((N_NODE, DO), jnp.float32),
    )(x, xp, csum, cmax, satt, cnt, amax, amin, w, b)


def _sc_side_body(key_hbm, oth_hbm, pk_hbm, po_hbm, xpo_hbm,
                  csum_o, cmax_o, satt_o, cnt_o, amax_o, amin_o,
                  svA, tvA, svB, tvB, pk_c, po_c, ls, lt, la,
                  acc_s, acc_m, rowbuf,
                  sa_t, cn_t, am_t, an_t, sem, sem_s, sem_s2):
    cid = lax.axis_index("c")
    sid = lax.axis_index("s")
    w = cid * NS + sid
    lo = w * RNG

    zero16 = jnp.zeros((16,), jnp.float32)
    ninf16 = jnp.full((16,), _NEG_INF, jnp.float32)
    pinf16 = jnp.full((16,), _POS_INF, jnp.float32)
    i16 = lax.iota(jnp.int32, 16)

    def _zacc(i, _):
        r = i // 8
        c = (i % 8) * 16
        acc_s[r, pl.ds(c, 16)] = zero16
        acc_m[r, pl.ds(c, 16)] = ninf16
        return 0
    lax.fori_loop(0, RNG * 8, _zacc, 0)

    def _zstat(i, _):
        sa_t[pl.ds(i * 16, 16)] = zero16
        cn_t[pl.ds(i * 16, 16)] = zero16
        am_t[pl.ds(i * 16, 16)] = ninf16
        an_t[pl.ds(i * 16, 16)] = pinf16
        return 0
    lax.fori_loop(0, RNG // 16, _zstat, 0)

    def _zlist(i, _):
        lt[pl.ds(i * 16, 16)] = jnp.zeros((16,), jnp.int32)
        return 0
    lax.fori_loop(0, LCAP // 16, _zlist, 0)

    base = w * N_NODE
    lob = lo + base

    def stage(ck, sv, tv):
        off = ck * C
        return (pltpu.async_copy(key_hbm.at[pl.ds(off, C)], sv, sem_s),
                pltpu.async_copy(oth_hbm.at[pl.ds(off, C)], tv, sem_s2))

    def chunk(ck, sv, tv, svn, tvn):
        pltpu.make_async_copy(key_hbm.at[pl.ds(0, C)], sv, sem_s).wait()
        pltpu.make_async_copy(oth_hbm.at[pl.ds(0, C)], tv, sem_s2).wait()

        @pl.when(ck + 1 < NCHUNK)
        def _():
            stage(ck + 1, svn, tvn)

        def bias(v, _):
            sv[pl.ds(v * 16, 16)] = sv[pl.ds(v * 16, 16)] + base
            tv[pl.ds(v * 16, 16)] = tv[pl.ds(v * 16, 16)] + base
            return 0
        lax.fori_loop(0, NV, bias, 0)

        pltpu.async_copy(pk_hbm.at[sv], pk_c, sem).wait()
        pltpu.async_copy(po_hbm.at[tv], po_c, sem).wait()

        def scan(v, ki):
            s16 = sv[pl.ds(v * 16, 16)]
            t16 = tv[pl.ds(v * 16, 16)]
            att = jnp.exp(-jnp.abs(pk_c[pl.ds(v * 16, 16)] +
                                   po_c[pl.ds(v * 16, 16)]))
            mi = (s16 >= lob) & (s16 < lob + RNG)
            plsc.store_compressed(ls.at[pl.ds(ki, 16)], s16 - lob, mask=mi)
            plsc.store_compressed(lt.at[pl.ds(ki, 16)], t16 - base, mask=mi)
            plsc.store_compressed(la.at[pl.ds(ki, 16)], att, mask=mi)
            return ki + jnp.sum(mi.astype(jnp.int32))

        ki = lax.fori_loop(0, NV, scan, 0)

        ls[pl.ds(ki, 16)] = jnp.zeros((16,), jnp.int32)
        lt[pl.ds(ki, 16)] = jnp.zeros((16,), jnp.int32)
        la[pl.ds(ki, 16)] = zero16
        kr = ((ki + 15) // 16) * 16

        nb = (kr + G - 1) // G

        def batch(i, _):
            b = i * G
            pltpu.async_copy(xpo_hbm.at[lt.at[pl.ds(b, G)]],
                             rowbuf, sem).wait()
            ng = jnp.minimum(G // 16, (kr - b + 15) // 16)

            def group(g, _):
                nv = ls[pl.ds(b + g * 16, 16)]
                av = la[pl.ds(b + g * 16, 16)]
                for k in range(16):
                    n = nv[k]
                    a = av[k]
                    j = g * 16 + k
                    for c in range(8):
                        rv = rowbuf[j, pl.ds(c * 16, 16)] * a
                        acc_s[n, pl.ds(c * 16, 16)] = (
                            acc_s[n, pl.ds(c * 16, 16)] + rv)
                        acc_m[n, pl.ds(c * 16, 16)] = jnp.maximum(
                            acc_m[n, pl.ds(c * 16, 16)], rv)
                    nb16 = (n // 16) * 16
                    msk = i16 == (n % 16)
                    vs = sa_t[pl.ds(nb16, 16)]
                    sa_t[pl.ds(nb16, 16)] = vs + jnp.where(msk, a, 0.0)
                    vc = cn_t[pl.ds(nb16, 16)]
                    cn_t[pl.ds(nb16, 16)] = vc + jnp.where(
                        msk & (a > 0.0), 1.0, 0.0)
                    va = am_t[pl.ds(nb16, 16)]
                    am_t[pl.ds(nb16, 16)] = jnp.where(
                        msk, jnp.maximum(va, a), va)
                    vi = an_t[pl.ds(nb16, 16)]
                    an_t[pl.ds(nb16, 16)] = jnp.where(
                        msk, jnp.minimum(vi, a), vi)
                return 0
            lax.fori_loop(0, ng, group, 0)
            return 0
        lax.fori_loop(0, nb, batch, 0)

    stage(0, svA, tvA)

    def chunk2(hk, _):
        chunk(2 * hk, svA, tvA, svB, tvB)
        chunk(2 * hk + 1, svB, tvB, svA, tvA)
        return 0
    lax.fori_loop(0, NCHUNK // 2, chunk2, 0)

    pltpu.sync_copy(acc_s, csum_o.at[pl.ds(lo, RNG)])
    pltpu.sync_copy(acc_m, cmax_o.at[pl.ds(lo, RNG)])
    pltpu.sync_copy(sa_t, satt_o.at[pl.ds(lo, RNG)])
    pltpu.sync_copy(cn_t, cnt_o.at[pl.ds(lo, RNG)])
    pltpu.sync_copy(am_t, amax_o.at[pl.ds(lo, RNG)])
    pltpu.sync_copy(an_t, amin_o.at[pl.ds(lo, RNG)])


def _sc_side(key_arr, oth_arr, p_key, p_oth, xp_oth):
    mesh = plsc.VectorSubcoreMesh(core_axis_name="c", subcore_axis_name="s")
    f32 = jnp.float32
    out_type = [
        jax.ShapeDtypeStruct((NP, F), f32),
        jax.ShapeDtypeStruct((NP, F), f32),
        jax.ShapeDtypeStruct((NP,), f32),
        jax.ShapeDtypeStruct((NP,), f32),
        jax.ShapeDtypeStruct((NP,), f32),
        jax.ShapeDtypeStruct((NP,), f32),
    ]
    scratch = [
        pltpu.VMEM((C,), jnp.int32),
        pltpu.VMEM((C,), jnp.int32),
        pltpu.VMEM((C,), jnp.int32),
        pltpu.VMEM((C,), jnp.int32),
        pltpu.VMEM((C,), f32),
        pltpu.VMEM((C,), f32),
        pltpu.VMEM((LCAP,), jnp.int32),
        pltpu.VMEM((LCAP,), jnp.int32),
        pltpu.VMEM((LCAP,), f32),
        pltpu.VMEM((RNG, F), f32),
        pltpu.VMEM((RNG, F), f32),
        pltpu.VMEM((G, F), f32),
        pltpu.VMEM((RNG,), f32),
        pltpu.VMEM((RNG,), f32),
        pltpu.VMEM((RNG,), f32),
        pltpu.VMEM((RNG,), f32),
        pltpu.SemaphoreType.DMA,
        pltpu.SemaphoreType.DMA,
        pltpu.SemaphoreType.DMA,
    ]
    fn = pl.kernel(_sc_side_body, out_type=out_type, mesh=mesh,
                   scratch_types=scratch,
                   compiler_params=pltpu.CompilerParams(
                       needs_layout_passes=False))
    return fn(key_arr, oth_arr, p_key, p_oth, xp_oth)


def kernel(x_intt_0, x_mvtx_0, x_intt, x_mvtx, edge_index,
           W_in_intt, b_in_intt, W_in_mvtx, b_in_mvtx,
           W_score, b_score, W_out_intt, b_out_intt,
           W_out_mvtx, b_out_mvtx):
    del x_intt_0, x_mvtx_0

    xp_i, p_i = _pre(x_intt, W_in_intt, b_in_intt.reshape(1, F),
                     W_score[:F], b_score.reshape(1, 1))
    xp_m, p_m = _pre(x_mvtx, W_in_mvtx, b_in_mvtx.reshape(1, F),
                     W_score[F:], jnp.zeros((1, 1), jnp.float32))

    s_arr = edge_index[0]
    t_arr = edge_index[1]
    p_i_v = jnp.tile(p_i[:, 0], NW)
    p_m_v = jnp.tile(p_m[:, 0], NW)

    csum_i, cmax_i, satt_i, cnt_i, amax_i, amin_i = _sc_side(
        s_arr, t_arr, p_i_v, p_m_v, xp_m)
    csum_m, cmax_m, satt_m, cnt_m, amax_m, amin_m = _sc_side(
        t_arr, s_arr, p_m_v, p_i_v, xp_i)

    col = lambda v: v[:N_NODE].reshape(N_NODE, 1)
    h_i = _post(x_intt, xp_i, csum_i[:N_NODE], cmax_i[:N_NODE],
                col(satt_i), col(cnt_i), col(amax_i), col(amin_i),
                W_out_intt, b_out_intt.reshape(1, DO), True)
    h_m = _post(x_mvtx, xp_m, csum_m[:N_NODE], cmax_m[:N_NODE],
                col(satt_m), col(cnt_m), col(amax_m), col(amin_m),
                W_out_mvtx, b_out_mvtx.reshape(1, DO), False)
    return (h_i, h_m)


# speedup vs baseline: 1.0361x; 1.0361x over previous
import functools

import jax
import jax.numpy as jnp
from jax import lax
from jax.experimental import pallas as pl
from jax.experimental.pallas import tpu as pltpu
from jax.experimental.pallas import tpu_sc as plsc

N_NODE = 10000
E = 160000
D = 128
F = 128
DO = 64

NC, NS = 2, 16
NW = NC * NS
RNG = 320
NP = NW * RNG
C = 800
NV = C // 16
NCHUNK = E // C
G = 32
LCAP = C + G + 16

_NEG_INF = float('-inf')
_POS_INF = float('inf')


def _pre_body(x_ref, w_ref, b_ref, ws_ref, bs_ref, xp_ref, p_ref):
    xp = jnp.dot(x_ref[...], w_ref[...], preferred_element_type=jnp.float32)
    xp = xp + b_ref[...]
    xp_ref[...] = xp
    p_ref[...] = jnp.dot(xp, ws_ref[...],
                         preferred_element_type=jnp.float32) + bs_ref[...]


def _pre(x, w, b, ws, bs):
    return pl.pallas_call(
        _pre_body,
        out_shape=(
            jax.ShapeDtypeStruct((N_NODE, F), jnp.float32),
            jax.ShapeDtypeStruct((N_NODE, 1), jnp.float32),
        ),
    )(x, w, b, ws, bs)


def _post_body(x_ref, xp_ref, csum_ref, cmax_ref, satt_ref, cnt_ref,
               amax_ref, amin_ref, w_ref, b_ref, h_ref, *, self_first):
    x = x_ref[...]
    xp = xp_ref[...]
    satt_ref_v = satt_ref[...]
    cnt = cnt_ref[...]
    amax_v = amax_ref[...]
    amin_v = amin_ref[...]
    has = cnt > 0.0
    inv = 1.0 / jnp.maximum(cnt, 1.0)
    mean_self = xp * (satt_ref_v * inv)
    mean_cross = csum_ref[...] * inv
    ms = jnp.where(xp >= 0.0, xp * amax_v, xp * amin_v)
    max_self = jnp.maximum(jnp.where(has, ms, 0.0), 0.0)
    max_cross = jnp.maximum(jnp.where(has, cmax_ref[...], 0.0), 0.0)
    if self_first:
        h = jnp.concatenate(
            [x, xp, mean_self, mean_cross, max_self, max_cross], axis=-1)
    else:
        h = jnp.concatenate(
            [x, xp, mean_cross, mean_self, max_cross, max_self], axis=-1)
    out = jnp.dot(h, w_ref[...], preferred_element_type=jnp.float32)
    h_ref[...] = jnp.maximum(out + b_ref[...], 0.0)


def _post(x, xp, csum, cmax, satt, cnt, amax, amin, w, b, self_first):
    blk = 2000
    grid = N_NODE // blk
    rs = lambda i: (i, 0)
    full = lambda i: (0, 0)
    return pl.pallas_call(
        functools.partial(_post_body, self_first=self_first),
        grid=(grid,),
        in_specs=[
            pl.BlockSpec((blk, D), rs),
            pl.BlockSpec((blk, F), rs),
            pl.BlockSpec((blk, F), rs),
            pl.BlockSpec((blk, F), rs),
            pl.BlockSpec((blk, 1), rs),
            pl.BlockSpec((blk, 1), rs),
            pl.BlockSpec((blk, 1), rs),
            pl.BlockSpec((blk, 1), rs),
            pl.BlockSpec((D + F + 4 * F, DO), full),
            pl.BlockSpec((1, DO), full),
        ],
        out_specs=pl.BlockSpec((blk, DO), rs),
        out_shape=jax.ShapeDtypeStruct((N_NODE, DO), jnp.float32),
    )(x, xp, csum, cmax, satt, cnt, amax, amin, w, b)


def _sc_side_body(key_hbm, oth_hbm, pk_hbm, po_hbm, xpo_hbm,
                  csum_o, cmax_o, satt_o, cnt_o, amax_o, amin_o,
                  sv, tv, pk_c, po_c, ls, lt, la,
                  acc_s, acc_m, rowbuf,
                  sa_t, cn_t, am_t, an_t, sem):
    cid = lax.axis_index("c")
    sid = lax.axis_index("s")
    w = cid * NS + sid
    lo = w * RNG

    zero16 = jnp.zeros((16,), jnp.float32)
    ninf16 = jnp.full((16,), _NEG_INF, jnp.float32)
    pinf16 = jnp.full((16,), _POS_INF, jnp.float32)
    i16 = lax.iota(jnp.int32, 16)

    def _zacc(i, _):
        r = i // 8
        c = (i % 8) * 16
        acc_s[r, pl.ds(c, 16)] = zero16
        acc_m[r, pl.ds(c, 16)] = ninf16
        return 0
    lax.fori_loop(0, RNG * 8, _zacc, 0)

    def _zstat(i, _):
        sa_t[pl.ds(i * 16, 16)] = zero16
        cn_t[pl.ds(i * 16, 16)] = zero16
        am_t[pl.ds(i * 16, 16)] = ninf16
        an_t[pl.ds(i * 16, 16)] = pinf16
        return 0
    lax.fori_loop(0, RNG // 16, _zstat, 0)

    def _zlist(i, _):
        lt[pl.ds(i * 16, 16)] = jnp.zeros((16,), jnp.int32)
        return 0
    lax.fori_loop(0, LCAP // 16, _zlist, 0)

    def chunk(ck, _):
        off = ck * C
        pltpu.sync_copy(key_hbm.at[pl.ds(off, C)], sv)
        pltpu.sync_copy(oth_hbm.at[pl.ds(off, C)], tv)
        pltpu.async_copy(pk_hbm.at[sv], pk_c, sem).wait()
        pltpu.async_copy(po_hbm.at[tv], po_c, sem).wait()

        def scan(v, ki):
            s16 = sv[pl.ds(v * 16, 16)]
            t16 = tv[pl.ds(v * 16, 16)]
            att = jnp.exp(-jnp.abs(pk_c[pl.ds(v * 16, 16)] +
                                   po_c[pl.ds(v * 16, 16)]))
            mi = (s16 >= lo) & (s16 < lo + RNG)
            plsc.store_compressed(ls.at[pl.ds(ki, 16)], s16 - lo, mask=mi)
            plsc.store_compressed(lt.at[pl.ds(ki, 16)], t16, mask=mi)
            plsc.store_compressed(la.at[pl.ds(ki, 16)], att, mask=mi)
            return ki + jnp.sum(mi.astype(jnp.int32))

        ki = lax.fori_loop(0, NV, scan, 0, unroll=2)

        ls[pl.ds(ki, 16)] = jnp.zeros((16,), jnp.int32)
        lt[pl.ds(ki, 16)] = jnp.zeros((16,), jnp.int32)
        la[pl.ds(ki, 16)] = zero16
        kr = ((ki + 15) // 16) * 16

        nb = (kr + G - 1) // G

        def batch(i, _):
            b = i * G
            pltpu.async_copy(xpo_hbm.at[lt.at[pl.ds(b, G)]],
                             rowbuf, sem).wait()
            ng = jnp.minimum(G // 16, (kr - b + 15) // 16)

            def group(g, _):
                nv = ls[pl.ds(b + g * 16, 16)]
                av = la[pl.ds(b + g * 16, 16)]
                for k in range(16):
                    n = nv[k]
                    a = av[k]
                    j = g * 16 + k
                    for c in range(8):
                        rv = rowbuf[j, pl.ds(c * 16, 16)] * a
                        acc_s[n, pl.ds(c * 16, 16)] = (
                            acc_s[n, pl.ds(c * 16, 16)] + rv)
                        acc_m[n, pl.ds(c * 16, 16)] = jnp.maximum(
                            acc_m[n, pl.ds(c * 16, 16)], rv)
                    nb16 = (n // 16) * 16
                    msk = i16 == (n % 16)
                    vs = sa_t[pl.ds(nb16, 16)]
                    sa_t[pl.ds(nb16, 16)] = vs + jnp.where(msk, a, 0.0)
                    vc = cn_t[pl.ds(nb16, 16)]
                    cn_t[pl.ds(nb16, 16)] = vc + jnp.where(
                        msk & (a > 0.0), 1.0, 0.0)
                    va = am_t[pl.ds(nb16, 16)]
                    am_t[pl.ds(nb16, 16)] = jnp.where(
                        msk, jnp.maximum(va, a), va)
                    vi = an_t[pl.ds(nb16, 16)]
                    an_t[pl.ds(nb16, 16)] = jnp.where(
                        msk, jnp.minimum(vi, a), vi)
                return 0
            lax.fori_loop(0, ng, group, 0)
            return 0
        lax.fori_loop(0, nb, batch, 0)
        return 0

    lax.fori_loop(0, NCHUNK, chunk, 0)

    pltpu.sync_copy(acc_s, csum_o.at[pl.ds(lo, RNG)])
    pltpu.sync_copy(acc_m, cmax_o.at[pl.ds(lo, RNG)])
    pltpu.sync_copy(sa_t, satt_o.at[pl.ds(lo, RNG)])
    pltpu.sync_copy(cn_t, cnt_o.at[pl.ds(lo, RNG)])
    pltpu.sync_copy(am_t, amax_o.at[pl.ds(lo, RNG)])
    pltpu.sync_copy(an_t, amin_o.at[pl.ds(lo, RNG)])


def _sc_side(key_arr, oth_arr, p_key, p_oth, xp_oth):
    mesh = plsc.VectorSubcoreMesh(core_axis_name="c", subcore_axis_name="s")
    f32 = jnp.float32
    out_type = [
        jax.ShapeDtypeStruct((NP, F), f32),
        jax.ShapeDtypeStruct((NP, F), f32),
        jax.ShapeDtypeStruct((NP,), f32),
        jax.ShapeDtypeStruct((NP,), f32),
        jax.ShapeDtypeStruct((NP,), f32),
        jax.ShapeDtypeStruct((NP,), f32),
    ]
    scratch = [
        pltpu.VMEM((C,), jnp.int32),
        pltpu.VMEM((C,), jnp.int32),
        pltpu.VMEM((C,), f32),
        pltpu.VMEM((C,), f32),
        pltpu.VMEM((LCAP,), jnp.int32),
        pltpu.VMEM((LCAP,), jnp.int32),
        pltpu.VMEM((LCAP,), f32),
        pltpu.VMEM((RNG, F), f32),
        pltpu.VMEM((RNG, F), f32),
        pltpu.VMEM((G, F), f32),
        pltpu.VMEM((RNG,), f32),
        pltpu.VMEM((RNG,), f32),
        pltpu.VMEM((RNG,), f32),
        pltpu.VMEM((RNG,), f32),
        pltpu.SemaphoreType.DMA,
    ]
    fn = pl.kernel(_sc_side_body, out_type=out_type, mesh=mesh,
                   scratch_types=scratch,
                   compiler_params=pltpu.CompilerParams(
                       needs_layout_passes=False))
    return fn(key_arr, oth_arr, p_key, p_oth, xp_oth)


def kernel(x_intt_0, x_mvtx_0, x_intt, x_mvtx, edge_index,
           W_in_intt, b_in_intt, W_in_mvtx, b_in_mvtx,
           W_score, b_score, W_out_intt, b_out_intt,
           W_out_mvtx, b_out_mvtx):
    del x_intt_0, x_mvtx_0

    xp_i, p_i = _pre(x_intt, W_in_intt, b_in_intt.reshape(1, F),
                     W_score[:F], b_score.reshape(1, 1))
    xp_m, p_m = _pre(x_mvtx, W_in_mvtx, b_in_mvtx.reshape(1, F),
                     W_score[F:], jnp.zeros((1, 1), jnp.float32))

    s_arr = edge_index[0]
    t_arr = edge_index[1]
    p_i_v = p_i[:, 0]
    p_m_v = p_m[:, 0]

    csum_i, cmax_i, satt_i, cnt_i, amax_i, amin_i = _sc_side(
        s_arr, t_arr, p_i_v, p_m_v, xp_m)
    csum_m, cmax_m, satt_m, cnt_m, amax_m, amin_m = _sc_side(
        t_arr, s_arr, p_m_v, p_i_v, xp_i)

    col = lambda v: v[:N_NODE].reshape(N_NODE, 1)
    h_i = _post(x_intt, xp_i, csum_i[:N_NODE], cmax_i[:N_NODE],
                col(satt_i), col(cnt_i), col(amax_i), col(amin_i),
                W_out_intt, b_out_intt.reshape(1, DO), True)
    h_m = _post(x_mvtx, xp_m, csum_m[:N_NODE], cmax_m[:N_NODE],
                col(satt_m), col(cnt_m), col(amax_m), col(amin_m),
                W_out_mvtx, b_out_mvtx.reshape(1, DO), False)
    return (h_i, h_m)
